# baseline (speedup 1.0000x reference)
import functools

import jax
import jax.numpy as jnp
from jax import lax
from jax.experimental import pallas as pl
from jax.experimental.pallas import tpu as pltpu
from jax.experimental.pallas import tpu_sc as plsc

PW = 4
NC, NS = 2, 16
NW = NC * NS
L = 16


def _tc_body(W, P, pos_ref, x_ref, invt_ref):
    x = x_ref[...]
    hw = x.shape[0]
    rowsq = jnp.sum(x * x, axis=1, keepdims=True)
    tmp = rowsq
    for j in range(1, PW):
        tmp = tmp + jnp.roll(rowsq, -j, axis=0)
    win = tmp
    for i in range(1, PW):
        win = win + jnp.roll(tmp, -i * W, axis=0)
    lane = lax.broadcasted_iota(jnp.int32, (P, hw), 1)
    oh = jnp.where(lane == pos_ref[...], 1.0, 0.0)
    ss = lax.dot_general(oh, win, (((1,), (0,)), ((), ())),
                         preferred_element_type=jnp.float32)
    inv = 1.0 / (jnp.sqrt(ss) + 1e-7)
    invt_ref[...] = jnp.broadcast_to(inv, (P, L))


def _sc_body(cdim, n_chunk_rows, cpw, chunk_base,
             table, idxt, invt, out, idx_v, inv_v, rows_v,
             sem_g0, sem_g1, sem_w):
    cid = lax.axis_index("c")
    sid = lax.axis_index("s")
    wid = sid * NC + cid
    ppc = n_chunk_rows // (PW * PW)
    ncc = cdim // L
    sem_g = (sem_g0, sem_g1)

    def start_chunk(c, buf):
        mloc = wid * cpw + c
        pltpu.sync_copy(idxt.at[mloc], idx_v.at[buf])
        pltpu.sync_copy(invt.at[pl.ds(mloc * ppc, ppc)], inv_v.at[buf])
        return pltpu.async_copy(table.at[idx_v.at[buf]], rows_v.at[buf],
                                sem_g[buf])

    gh = [None, None]
    wh = [None, None]
    gh[0] = start_chunk(0, 0)
    for c in range(cpw):
        buf = c % 2
        nbuf = (c + 1) % 2
        if c + 1 < cpw:
            if wh[nbuf] is not None:
                for h in wh[nbuf]:
                    h.wait()
                wh[nbuf] = None
            gh[nbuf] = start_chunk(c + 1, nbuf)
        gh[buf].wait()
        for pi in range(ppc):
            inv = inv_v[buf, pi, pl.ds(0, L)]

            def scale_row(i, carry2, _buf=buf, _pi=pi, _inv=inv):
                rw = i * ppc + _pi
                for cc in range(ncc):
                    sl = pl.ds(cc * L, L)
                    rows_v[_buf, rw, sl] = rows_v[_buf, rw, sl] * _inv
                return carry2
            lax.fori_loop(0, PW * PW, scale_row, 0)
        mglob = chunk_base + wid * cpw + c
        wh[buf] = [pltpu.async_copy(
            rows_v.at[buf].at[pl.ds(g * ppc, ppc)],
            out.at[pl.ds(mglob * ppc, ppc), pl.ds(g * cdim, cdim)],
            sem_w) for g in range(PW * PW)]
    for b in range(2):
        if wh[b] is not None:
            for h in wh[b]:
                h.wait()


def kernel(feats, num_patches, patch_ids):
    B, C, H, W = feats.shape
    P = patch_ids.shape[0]
    hw = H * W
    D = PW * PW * C

    table = jnp.transpose(feats, (0, 2, 3, 1)).reshape(B * hw, C)

    r = patch_ids[:, 0].astype(jnp.int32)
    c = patch_ids[:, 1].astype(jnp.int32)
    pos = (r * W + c).reshape(P, 1)
    k = jnp.arange(PW * PW, dtype=jnp.int32)
    offs = (k // PW) * W + (k % PW)
    idx = (jnp.arange(B, dtype=jnp.int32) * hw)[:, None, None] \
        + pos[None, :, :] + offs[None, None, :]

    total_rows = B * P * PW * PW
    n_chunk_rows = 128
    n_chunks = total_rows // n_chunk_rows
    ppc = n_chunk_rows // (PW * PW)
    idxt = idx.reshape(n_chunks, ppc, PW * PW).transpose(0, 2, 1) \
              .reshape(n_chunks, n_chunk_rows)

    chunks_per_b = n_chunks // B

    mesh = plsc.VectorSubcoreMesh(core_axis_name="c", subcore_axis_name="s")
    out_ref = jax.new_ref(lax.empty((B * P, D), jnp.float32))

    parts = ((0, 1), (1, B - 1))
    for b0, nb in parts:
        invt_h = pl.pallas_call(
            functools.partial(_tc_body, W, P),
            grid=(nb,),
            in_specs=[
                pl.BlockSpec((P, 1), lambda b: (0, 0)),
                pl.BlockSpec((hw, C), lambda b, _b0=b0: (b + _b0, 0)),
            ],
            out_specs=pl.BlockSpec((P, L), lambda b: (b, 0)),
            out_shape=jax.ShapeDtypeStruct((nb * P, L), jnp.float32),
        )(pos, table)

        chunk_base = b0 * chunks_per_b
        n_chunks_part = nb * chunks_per_b
        cpw = n_chunks_part // NW
        idxt_h = lax.slice_in_dim(idxt, chunk_base,
                                  chunk_base + n_chunks_part, axis=0)

        sc_call = pl.kernel(
            functools.partial(_sc_body, C, n_chunk_rows, cpw, chunk_base),
            out_type=(),
            mesh=mesh,
            scratch_types=[
                pltpu.VMEM((2, n_chunk_rows), jnp.int32),
                pltpu.VMEM((2, ppc, L), jnp.float32),
                pltpu.VMEM((2, n_chunk_rows, C), jnp.float32),
                pltpu.SemaphoreType.DMA,
                pltpu.SemaphoreType.DMA,
                pltpu.SemaphoreType.DMA,
            ],
        )
        sc_call(table, idxt_h, invt_h, out_ref)

    out = out_ref[...]
    return (out, patch_ids)


# speedup vs baseline: 1.0415x; 1.0415x over previous
import functools

import jax
import jax.numpy as jnp
from jax import lax
from jax.experimental import pallas as pl
from jax.experimental.pallas import tpu as pltpu
from jax.experimental.pallas import tpu_sc as plsc

PW = 4
NC, NS = 2, 16
NW = NC * NS
L = 16


def _tc_body(W, P, pos_ref, x_ref, invt_ref):
    x = x_ref[...]
    hw = x.shape[0]
    rowsq = jnp.sum(x * x, axis=1, keepdims=True)
    tmp = rowsq
    for j in range(1, PW):
        tmp = tmp + jnp.roll(rowsq, -j, axis=0)
    win = tmp
    for i in range(1, PW):
        win = win + jnp.roll(tmp, -i * W, axis=0)
    lane = lax.broadcasted_iota(jnp.int32, (P, hw), 1)
    oh = jnp.where(lane == pos_ref[...], 1.0, 0.0)
    ss = lax.dot_general(oh, win, (((1,), (0,)), ((), ())),
                         preferred_element_type=jnp.float32)
    inv = 1.0 / (jnp.sqrt(ss) + 1e-7)
    invt_ref[...] = jnp.broadcast_to(inv, (P, L))


def _sc_body(cdim, n_chunk_rows, cpw, chunk_base,
             table, idxt, invt, out, idx_v, inv_v, rows_v,
             sem_g0, sem_g1, sem_w):
    cid = lax.axis_index("c")
    sid = lax.axis_index("s")
    wid = sid * NC + cid
    ppc = n_chunk_rows // (PW * PW)
    ncc = cdim // L
    sem_g = (sem_g0, sem_g1)

    def start_chunk(c, buf):
        mloc = wid * cpw + c
        pltpu.sync_copy(idxt.at[mloc], idx_v.at[buf])
        pltpu.sync_copy(invt.at[pl.ds(mloc * ppc, ppc)], inv_v.at[buf])
        return pltpu.async_copy(table.at[idx_v.at[buf]], rows_v.at[buf],
                                sem_g[buf])

    gh = [None, None]
    wh = [None, None]
    gh[0] = start_chunk(0, 0)
    for c in range(cpw):
        buf = c % 2
        nbuf = (c + 1) % 2
        if c + 1 < cpw:
            if wh[nbuf] is not None:
                for h in wh[nbuf]:
                    h.wait()
                wh[nbuf] = None
            gh[nbuf] = start_chunk(c + 1, nbuf)
        gh[buf].wait()
        for pi in range(ppc):
            inv = inv_v[buf, pi, pl.ds(0, L)]

            def scale_row(i, carry2, _buf=buf, _pi=pi, _inv=inv):
                rw = i * ppc + _pi
                for cc in range(ncc):
                    sl = pl.ds(cc * L, L)
                    rows_v[_buf, rw, sl] = rows_v[_buf, rw, sl] * _inv
                return carry2
            lax.fori_loop(0, PW * PW, scale_row, 0)
        mglob = chunk_base + wid * cpw + c
        wh[buf] = [pltpu.async_copy(
            rows_v.at[buf].at[pl.ds(g * ppc, ppc)],
            out.at[pl.ds(mglob * ppc, ppc), pl.ds(g * cdim, cdim)],
            sem_w) for g in range(PW * PW)]
    for b in range(2):
        if wh[b] is not None:
            for h in wh[b]:
                h.wait()


def kernel(feats, num_patches, patch_ids):
    B, C, H, W = feats.shape
    P = patch_ids.shape[0]
    hw = H * W
    D = PW * PW * C

    table = jnp.transpose(feats, (0, 2, 3, 1)).reshape(B * hw, C)

    r = patch_ids[:, 0].astype(jnp.int32)
    c = patch_ids[:, 1].astype(jnp.int32)
    pos = (r * W + c).reshape(P, 1)
    k = jnp.arange(PW * PW, dtype=jnp.int32)
    offs = (k // PW) * W + (k % PW)
    idx = (jnp.arange(B, dtype=jnp.int32) * hw)[:, None, None] \
        + pos[None, :, :] + offs[None, None, :]

    total_rows = B * P * PW * PW
    n_chunk_rows = 128
    n_chunks = total_rows // n_chunk_rows
    ppc = n_chunk_rows // (PW * PW)
    idxt = idx.reshape(n_chunks, ppc, PW * PW).transpose(0, 2, 1) \
              .reshape(n_chunks, n_chunk_rows)

    chunks_per_b = n_chunks // B

    mesh = plsc.VectorSubcoreMesh(core_axis_name="c", subcore_axis_name="s")
    out_ref = jax.new_ref(lax.empty((B * P, D), jnp.float32))

    parts = ((0, B),)
    for b0, nb in parts:
        invt_h = pl.pallas_call(
            functools.partial(_tc_body, W, P),
            grid=(nb,),
            in_specs=[
                pl.BlockSpec((P, 1), lambda b: (0, 0)),
                pl.BlockSpec((hw, C), lambda b, _b0=b0: (b + _b0, 0)),
            ],
            out_specs=pl.BlockSpec((P, L), lambda b: (b, 0)),
            out_shape=jax.ShapeDtypeStruct((nb * P, L), jnp.float32),
        )(pos, table)

        chunk_base = b0 * chunks_per_b
        n_chunks_part = nb * chunks_per_b
        cpw = n_chunks_part // NW
        idxt_h = lax.slice_in_dim(idxt, chunk_base,
                                  chunk_base + n_chunks_part, axis=0)

        sc_call = pl.kernel(
            functools.partial(_sc_body, C, n_chunk_rows, cpw, chunk_base),
            out_type=(),
            mesh=mesh,
            scratch_types=[
                pltpu.VMEM((2, n_chunk_rows), jnp.int32),
                pltpu.VMEM((2, ppc, L), jnp.float32),
                pltpu.VMEM((2, n_chunk_rows, C), jnp.float32),
                pltpu.SemaphoreType.DMA,
                pltpu.SemaphoreType.DMA,
                pltpu.SemaphoreType.DMA,
            ],
        )
        sc_call(table, idxt_h, invt_h, out_ref)

    out = out_ref[...]
    return (out, patch_ids)
